# baseline (speedup 1.0000x reference)
import functools

import jax
import jax.numpy as jnp
from jax import lax
from jax.experimental import pallas as pl
from jax.experimental.pallas import tpu as pltpu
from jax.experimental.pallas import tpu_sc as plsc

_K = 32
_B = 16384
_NC = 2
_NS = 16
_NW = _NC * _NS
_BPW = _B // _NW

_BLK = 2048


_sc_mesh = plsc.VectorSubcoreMesh(core_axis_name="c", subcore_axis_name="s")


@functools.partial(
    pl.kernel,
    mesh=_sc_mesh,
    compiler_params=pltpu.CompilerParams(use_tc_tiling_on_sc=False),
    out_type=[jax.ShapeDtypeStruct((_B, _K), jnp.float32)] * 4,
    scratch_types=[
        pltpu.VMEM((_BPW,), jnp.int32),
        pltpu.VMEM((_BPW,), jnp.int32),
        pltpu.VMEM((_BPW,), jnp.int32),
        pltpu.VMEM((_BPW,), jnp.int32),
        pltpu.VMEM((_BPW, _K), jnp.float32),
        pltpu.VMEM((_BPW, _K), jnp.float32),
        pltpu.VMEM((_BPW, _K), jnp.float32),
        pltpu.VMEM((_BPW, _K), jnp.float32),
        pltpu.SemaphoreType.DMA,
    ],
)
def _sc_gather(user_hbm, item_hbm, tag_hbm, uid_hbm, iid_hbm, pid_hbm, nid_hbm,
               uo, io, to, no,
               uidx, iidx, pidx, nidx, ubuf, ibuf, tbuf, nbuf, sem):
    wid = lax.axis_index("s") * _NC + lax.axis_index("c")
    base = wid * _BPW
    sl = pl.ds(base, _BPW)
    pltpu.sync_copy(uid_hbm.at[sl], uidx)
    pltpu.sync_copy(iid_hbm.at[sl], iidx)
    pltpu.sync_copy(pid_hbm.at[sl], pidx)
    pltpu.sync_copy(nid_hbm.at[sl], nidx)
    cu = pltpu.async_copy(user_hbm.at[uidx], ubuf, sem)
    ci = pltpu.async_copy(item_hbm.at[iidx], ibuf, sem)
    ct = pltpu.async_copy(tag_hbm.at[pidx], tbuf, sem)
    cn = pltpu.async_copy(tag_hbm.at[nidx], nbuf, sem)
    cu.wait()
    ci.wait()
    ct.wait()
    cn.wait()
    pltpu.sync_copy(ubuf, uo.at[sl])
    pltpu.sync_copy(ibuf, io.at[sl])
    pltpu.sync_copy(tbuf, to.at[sl])
    pltpu.sync_copy(nbuf, no.at[sl])


def _tc_dense_body(uv, iv, tv, nv, wut, wit, bu, bi, out):
    t = tv[...]
    n = nv[...]
    wu = wut[...]
    wi = wit[...]
    ut = jax.nn.sigmoid(jnp.dot(t, wu, preferred_element_type=jnp.float32) + bu[...])
    it = jax.nn.sigmoid(jnp.dot(t, wi, preferred_element_type=jnp.float32) + bi[...])
    nu = jax.nn.sigmoid(jnp.dot(n, wu, preferred_element_type=jnp.float32) + bu[...])
    ni = jax.nn.sigmoid(jnp.dot(n, wi, preferred_element_type=jnp.float32) + bi[...])
    out[...] = jnp.sum(uv[...] * (ut - nu) + iv[...] * (it - ni), axis=1)


_tc_dense = pl.pallas_call(
    _tc_dense_body,
    grid=(_B // _BLK,),
    in_specs=[
        pl.BlockSpec((_BLK, _K), lambda i: (i, 0)),
        pl.BlockSpec((_BLK, _K), lambda i: (i, 0)),
        pl.BlockSpec((_BLK, _K), lambda i: (i, 0)),
        pl.BlockSpec((_BLK, _K), lambda i: (i, 0)),
        pl.BlockSpec((_K, _K), lambda i: (0, 0)),
        pl.BlockSpec((_K, _K), lambda i: (0, 0)),
        pl.BlockSpec((1, _K), lambda i: (0, 0)),
        pl.BlockSpec((1, _K), lambda i: (0, 0)),
    ],
    out_specs=pl.BlockSpec((_BLK,), lambda i: (i,)),
    out_shape=jax.ShapeDtypeStruct((_B,), jnp.float32),
)


def kernel(x, userVecs, itemVecs, tagVecs, Wu, bu, Wi, bi):
    if x.ndim == 1:
        x = x.reshape(1, -1)
    xi = x.astype(jnp.int32)
    uid, iid, pid, nid = xi[:, 0], xi[:, 1], xi[:, 2], xi[:, 3]
    uv, iv, tv, nv = _sc_gather(userVecs, itemVecs, tagVecs, uid, iid, pid, nid)
    return _tc_dense(uv, iv, tv, nv, Wu.T, Wi.T, bu[None, :], bi[None, :])
